# baseline (speedup 1.0000x reference)
import functools

import jax
import jax.numpy as jnp
from jax import lax
from jax.experimental import pallas as pl
from jax.experimental.pallas import tpu as pltpu
from jax.experimental.pallas import tpu_sc as plsc

_N = 10000
_E = 320000
_D = 128
_H = 64

_CH = 128
_NPAD = 10240
_AGG_ROWS = 10112
_HIST_ROWS = 20096
_EPAD = 327680
_DEGPAD = 655360

_AGG_TPT = _AGG_ROWS // 16
_HIST_TPT = _HIST_ROWS // 16

_HIGH = jax.lax.Precision.HIGHEST


def _embed_deg_body(ids_hbm, didx_hbm, emb_hbm,
                    h0_hbm, histp_hbm,
                    idx_v, rows_v, didx_v, hist1d, gsem):
    c = lax.axis_index("c")
    s = lax.axis_index("s")
    wid = s * 2 + c

    pltpu.sync_copy(didx_hbm.at[pl.ds((c * 16 + s) * 20480, 20480)], didx_v)

    zero16 = jnp.zeros((16,), jnp.float32)

    def _z(r, carry):
        hist1d[pl.ds(r * 16, 16)] = zero16
        return carry

    lax.fori_loop(0, _HIST_ROWS // 16, _z, 0)

    for t in range(3):
        g = wid + 32 * t

        @pl.when(g < _NPAD // _CH)
        def _():
            pltpu.sync_copy(ids_hbm.at[g], idx_v)
            pltpu.async_copy(emb_hbm.at[idx_v], rows_v, gsem).wait()
            pltpu.sync_copy(rows_v, h0_hbm.at[pl.ds(g * _CH, _CH)])

    ones16 = jnp.ones((16,), jnp.float32)

    def _deg(i, carry):
        idxv = didx_v[pl.ds(i * 16, 16)]
        plsc.addupdate_scatter(hist1d, [idxv], ones16)
        return carry

    lax.fori_loop(0, 20480 // 16, _deg, 0)
    pltpu.sync_copy(hist1d, histp_hbm.at[wid])


@functools.lru_cache(maxsize=None)
def _sc_mesh():
    return plsc.VectorSubcoreMesh(core_axis_name="c", subcore_axis_name="s")


def _embed_deg(*args):
    return pl.kernel(
        _embed_deg_body,
        out_type=(jax.ShapeDtypeStruct((_NPAD, _D), jnp.float32),
                  jax.ShapeDtypeStruct((32, _HIST_ROWS), jnp.float32)),
        mesh=_sc_mesh(),
        scratch_types=[
            pltpu.VMEM((_CH,), jnp.int32),
            pltpu.VMEM((_CH, _D), jnp.float32),
            pltpu.VMEM((20480,), jnp.int32),
            pltpu.VMEM((_HIST_ROWS,), jnp.float32),
            pltpu.SemaphoreType.DMA,
        ],
        compiler_params=pltpu.CompilerParams(needs_layout_passes=False),
    )(*args)


def _scatter_body(ml_hbm, mr_hbm, src_hbm, dst_hbm,
                  aggl_hbm, aggr_hbm,
                  idx_s, idx_d, rows_v, agg_sh, gsems, ssems):
    c = lax.axis_index("c")
    s = lax.axis_index("s")

    zero16 = jnp.zeros((16,), jnp.float32)
    zbuf = rows_v.at[0]

    def _z(r, carry):
        for k in range(4):
            zbuf[r, pl.ds(k * 16, 16)] = zero16
        return carry

    lax.fori_loop(0, _CH, _z, 0)
    for k in range(4):
        pltpu.sync_copy(zbuf,
                        agg_sh.at[pl.ds(s * _AGG_TPT + k * _CH, _CH)])
    pltpu.sync_copy(zbuf.at[pl.ds(0, 120)],
                    agg_sh.at[pl.ds(s * _AGG_TPT + 512, 120)])
    pltpu.sync_copy(src_hbm.at[pl.ds(s * 160, 160)], idx_s)
    pltpu.sync_copy(dst_hbm.at[pl.ds(s * 160, 160)], idx_d)
    plsc.subcore_barrier()

    def _run(m_hbm):
        def _gwait(b, sem):
            pltpu.make_async_copy(m_hbm.at[idx_s.at[0]],
                                  rows_v.at[b], sem).wait()

        def _swait(b, sem):
            pltpu.make_async_copy(rows_v.at[b],
                                  agg_sh.at[idx_d.at[0]], sem).wait()

        for b in range(4):
            pltpu.async_copy(m_hbm.at[idx_s.at[b]], rows_v.at[b], gsems[b])

        def _edge(t, carry):
            g0 = t * 4
            for b in range(4):
                _gwait(b, gsems[b])
                pltpu.async_copy(rows_v.at[b], agg_sh.at[idx_d.at[g0 + b]],
                                 ssems[b], add=True)
            for b in range(4):
                @pl.when(g0 + b + 4 < 160)
                def _(b=b):
                    _swait(b, ssems[b])
                    pltpu.async_copy(m_hbm.at[idx_s.at[g0 + b + 4]],
                                     rows_v.at[b], gsems[b])
            return carry

        lax.fori_loop(0, 40, _edge, 0)
        for b in range(4):
            _swait(b, ssems[b])

    @pl.when(c == 0)
    def _():
        _run(ml_hbm)

    @pl.when(c == 1)
    def _():
        _run(mr_hbm)

    plsc.subcore_barrier()

    def _wb(out_hbm):
        for k in range(4):
            buf = rows_v.at[k]
            pltpu.sync_copy(agg_sh.at[pl.ds(s * _AGG_TPT + k * _CH, _CH)],
                            buf)
            pltpu.sync_copy(buf, out_hbm.at[pl.ds(s * _AGG_TPT + k * _CH,
                                                  _CH)])
        pltpu.sync_copy(agg_sh.at[pl.ds(s * _AGG_TPT + 512, 120)],
                        rows_v.at[0].at[pl.ds(0, 120)])
        pltpu.sync_copy(rows_v.at[0].at[pl.ds(0, 120)],
                        out_hbm.at[pl.ds(s * _AGG_TPT + 512, 120)])

    @pl.when(c == 0)
    def _():
        _wb(aggl_hbm)

    @pl.when(c == 1)
    def _():
        _wb(aggr_hbm)


def _sc_scatter(*args):
    return pl.kernel(
        _scatter_body,
        out_type=(jax.ShapeDtypeStruct((_AGG_ROWS, 64), jnp.float32),
                  jax.ShapeDtypeStruct((_AGG_ROWS, 64), jnp.float32)),
        mesh=_sc_mesh(),
        scratch_types=[
            pltpu.VMEM((160, _CH), jnp.int32),
            pltpu.VMEM((160, _CH), jnp.int32),
            pltpu.VMEM((4, _CH, 64), jnp.float32),
            pltpu.VMEM_SHARED((_AGG_ROWS, 64), jnp.float32),
            [pltpu.SemaphoreType.DMA] * 4,
            [pltpu.SemaphoreType.DMA] * 4,
        ],
        compiler_params=pltpu.CompilerParams(needs_layout_passes=False,
                                             use_tc_tiling_on_sc=False),
    )(*args)


_BR = 400


def _t0_body(hp_ref, norm_ref):
    deg = jnp.sum(hp_ref[...], axis=0)
    norm_ref[...] = lax.rsqrt(jnp.maximum(deg, 1.0))


def _t0(histp3):
    return pl.pallas_call(
        _t0_body,
        grid=(1,),
        in_specs=[pl.BlockSpec((32, _HIST_ROWS // _D, _D),
                               lambda i: (0, 0, 0))],
        out_specs=pl.BlockSpec((_HIST_ROWS // _D, _D), lambda i: (0, 0)),
        out_shape=jax.ShapeDtypeStruct((_HIST_ROWS // _D, _D), jnp.float32),
    )(histp3)


def _t1_body(h0_ref, no_ref, w_ref, ml_ref, mr_ref):
    hw = jnp.dot(h0_ref[...], w_ref[...],
                 preferred_element_type=jnp.float32, precision=_HIGH)
    m = hw * no_ref[...]
    ml_ref[...] = m[:, :64]
    mr_ref[...] = m[:, 64:]


_half_out_specs = [
    pl.BlockSpec((_BR, 64), lambda i: (i, 0)),
    pl.BlockSpec((_BR, 64), lambda i: (i, 0)),
]
_half_out_shape = [
    jax.ShapeDtypeStruct((_N, 64), jnp.float32),
    jax.ShapeDtypeStruct((_N, 64), jnp.float32),
]


def _t1(h0, normo, w):
    return pl.pallas_call(
        _t1_body,
        grid=(_N // _BR,),
        in_specs=[
            pl.BlockSpec((_BR, _D), lambda i: (i, 0)),
            pl.BlockSpec((_BR, 1), lambda i: (i, 0)),
            pl.BlockSpec((_D, _D), lambda i: (0, 0)),
        ],
        out_specs=_half_out_specs,
        out_shape=_half_out_shape,
    )(h0, normo, w)


def _t2_body(pl_ref, pr_ref, ni_ref, no_ref, b_ref, w_ref, ml_ref, mr_ref):
    agg = jnp.concatenate([pl_ref[...], pr_ref[...]], axis=-1)
    h1 = agg * ni_ref[...] + b_ref[...]
    hw = jnp.dot(h1, w_ref[...],
                 preferred_element_type=jnp.float32, precision=_HIGH)
    m = hw * no_ref[...]
    ml_ref[...] = m[:, :64]
    mr_ref[...] = m[:, 64:]


def _t2(aggl, aggr, ni, no, b2d, w):
    return pl.pallas_call(
        _t2_body,
        grid=(_N // _BR,),
        in_specs=[
            pl.BlockSpec((_BR, 64), lambda i: (i, 0)),
            pl.BlockSpec((_BR, 64), lambda i: (i, 0)),
            pl.BlockSpec((_BR, 1), lambda i: (i, 0)),
            pl.BlockSpec((_BR, 1), lambda i: (i, 0)),
            pl.BlockSpec((1, _D), lambda i: (0, 0)),
            pl.BlockSpec((_D, _D), lambda i: (0, 0)),
        ],
        out_specs=_half_out_specs,
        out_shape=_half_out_shape,
    )(aggl, aggr, ni, no, b2d, w)


def _ln(x, g, b):
    mu = jnp.mean(x, axis=-1, keepdims=True)
    var = jnp.mean((x - mu) ** 2, axis=-1, keepdims=True)
    return (x - mu) * lax.rsqrt(var + 1e-5) * g + b


def _t3_body(pl_ref, pr_ref, ni_ref, b_ref, g1_ref, b1_ref, w2_ref, b2_ref,
             g2_ref, b2b_ref, w3_ref, b3_ref, out_ref, acc):
    i = pl.program_id(0)
    agg = jnp.concatenate([pl_ref[...], pr_ref[...]], axis=-1)
    h2 = agg * ni_ref[...] + b_ref[...]
    bm = jnp.max(h2, axis=0, keepdims=True)

    @pl.when(i == 0)
    def _():
        acc[...] = jnp.full((8, _D), -jnp.inf, jnp.float32)

    acc[...] = jnp.maximum(acc[...], jnp.broadcast_to(bm, (8, _D)))

    @pl.when(i == pl.num_programs(0) - 1)
    def _():
        x = _ln(acc[0:1, :], g1_ref[...], b1_ref[...])
        y = jnp.dot(x, w2_ref[...],
                    preferred_element_type=jnp.float32, precision=_HIGH)
        y = jnp.maximum(y + b2_ref[...], 0.0)
        y = _ln(y, g2_ref[...], b2b_ref[...])
        out_ref[...] = (jnp.sum(y * w3_ref[...], axis=-1, keepdims=True)
                        + b3_ref[...])


def _t3(aggl, aggr, ni, b2d, g1, b1, w2, b2, g2, b2b, w3t, b3p):
    def full(shape):
        return pl.BlockSpec(shape, lambda i: tuple(0 for _ in shape))

    return pl.pallas_call(
        _t3_body,
        grid=(_N // _BR,),
        in_specs=[
            pl.BlockSpec((_BR, 64), lambda i: (i, 0)),
            pl.BlockSpec((_BR, 64), lambda i: (i, 0)),
            pl.BlockSpec((_BR, 1), lambda i: (i, 0)),
            full((1, _D)), full((1, _D)), full((1, _D)),
            full((_D, _H)), full((1, _H)), full((1, _H)), full((1, _H)),
            full((1, _H)), full((1, _D)),
        ],
        out_specs=pl.BlockSpec((1, _D), lambda i: (0, 0)),
        out_shape=jax.ShapeDtypeStruct((1, _D), jnp.float32),
        scratch_shapes=[pltpu.VMEM((8, _D), jnp.float32)],
        compiler_params=pltpu.CompilerParams(
            dimension_semantics=("arbitrary",)),
    )(aggl, aggr, ni, b2d, g1, b1, w2, b2, g2, b2b, w3t, b3p)


def kernel(node_ids, edge_index, emb, W, b, ln1_g, ln1_b, W2, b2,
           ln2_g, ln2_b, W3, b3):
    node_ids = node_ids.astype(jnp.int32)
    src = edge_index[0].astype(jnp.int32)
    dst = edge_index[1].astype(jnp.int32)

    ids2d = jnp.concatenate(
        [node_ids, jnp.zeros((_NPAD - _N,), jnp.int32)]).reshape(-1, _CH)
    src2d = jnp.concatenate(
        [src, jnp.zeros((_EPAD - _E,), jnp.int32)]).reshape(-1, _CH)
    dst2d = jnp.concatenate(
        [dst, jnp.full((_EPAD - _E,), _N, jnp.int32)]).reshape(-1, _CH)
    didx = jnp.concatenate(
        [src, dst + _N,
         jnp.full((_DEGPAD - 2 * _E,), 2 * _N, jnp.int32)])

    h0p, histp = _embed_deg(ids2d, didx, emb)
    norm = _t0(histp.reshape(32, _HIST_ROWS // _D, _D))
    nf = norm.reshape(_HIST_ROWS)
    normo = nf[:_N].reshape(_N, 1)
    normi = nf[_N:2 * _N].reshape(_N, 1)

    m1l, m1r = _t1(h0p, normo, W)
    p1l, p1r = _sc_scatter(m1l, m1r, src2d, dst2d)
    b2d = b.reshape(1, _D)
    m2l, m2r = _t2(p1l, p1r, normi, normo, b2d, W)
    p2l, p2r = _sc_scatter(m2l, m2r, src2d, dst2d)
    outp = _t3(p2l, p2r, normi, b2d,
               ln1_g.reshape(1, _D), ln1_b.reshape(1, _D),
               W2, b2.reshape(1, _H),
               ln2_g.reshape(1, _H), ln2_b.reshape(1, _H),
               W3.reshape(1, _H),
               jnp.broadcast_to(b3.reshape(1, 1), (1, _D)))
    return outp[0:1, 0:1]


# speedup vs baseline: 1.5218x; 1.5218x over previous
import functools

import jax
import jax.numpy as jnp
from jax import lax
from jax.experimental import pallas as pl
from jax.experimental.pallas import tpu as pltpu
from jax.experimental.pallas import tpu_sc as plsc

_N = 10000
_E = 320000
_D = 128
_H = 64

_CH = 128
_NPAD = 10240
_AGG_ROWS = 10112
_HIST_ROWS = 20096
_EPAD = 327680
_DEGPAD = 655360

_AGG_TPT = _AGG_ROWS // 16
_HIST_TPT = _HIST_ROWS // 16

_HIGH = jax.lax.Precision.HIGHEST


def _embed_deg_body(ids_hbm, didx_hbm, emb_hbm,
                    h0_hbm, histp_hbm,
                    idx_v, rows_v, didx_v, hist1d, gsem):
    c = lax.axis_index("c")
    s = lax.axis_index("s")
    wid = s * 2 + c

    pltpu.sync_copy(didx_hbm.at[pl.ds((c * 16 + s) * 20480, 20480)], didx_v)

    zero16 = jnp.zeros((16,), jnp.float32)

    def _z(r, carry):
        hist1d[pl.ds(r * 16, 16)] = zero16
        return carry

    lax.fori_loop(0, _HIST_ROWS // 16, _z, 0)

    for t in range(3):
        g = wid + 32 * t

        @pl.when(g < _NPAD // _CH)
        def _():
            pltpu.sync_copy(ids_hbm.at[g], idx_v)
            pltpu.async_copy(emb_hbm.at[idx_v], rows_v, gsem).wait()
            pltpu.sync_copy(rows_v, h0_hbm.at[pl.ds(g * _CH, _CH)])

    ones16 = jnp.ones((16,), jnp.float32)

    def _deg(i, carry):
        idxv = didx_v[pl.ds(i * 16, 16)]
        plsc.addupdate_scatter(hist1d, [idxv], ones16)
        return carry

    lax.fori_loop(0, 20480 // 16, _deg, 0)
    pltpu.sync_copy(hist1d, histp_hbm.at[wid])


@functools.lru_cache(maxsize=None)
def _sc_mesh():
    return plsc.VectorSubcoreMesh(core_axis_name="c", subcore_axis_name="s")


def _embed_deg(*args):
    return pl.kernel(
        _embed_deg_body,
        out_type=(jax.ShapeDtypeStruct((_NPAD, _D), jnp.float32),
                  jax.ShapeDtypeStruct((32, _HIST_ROWS), jnp.float32)),
        mesh=_sc_mesh(),
        scratch_types=[
            pltpu.VMEM((_CH,), jnp.int32),
            pltpu.VMEM((_CH, _D), jnp.float32),
            pltpu.VMEM((20480,), jnp.int32),
            pltpu.VMEM((_HIST_ROWS,), jnp.float32),
            pltpu.SemaphoreType.DMA,
        ],
        compiler_params=pltpu.CompilerParams(needs_layout_passes=False),
    )(*args)


def _scatter_body(ml_hbm, mr_hbm, src_hbm, dst_hbm,
                  aggl_hbm, aggr_hbm,
                  idx_s, idx_d, rows_v, m_sh, agg_sh, gsems, ssems):
    c = lax.axis_index("c")
    s = lax.axis_index("s")

    zero16 = jnp.zeros((16,), jnp.float32)
    zbuf = rows_v.at[0]

    def _z(r, carry):
        for k in range(4):
            zbuf[r, pl.ds(k * 16, 16)] = zero16
        return carry

    lax.fori_loop(0, _CH, _z, 0)
    for k in range(4):
        pltpu.sync_copy(zbuf,
                        agg_sh.at[pl.ds(s * _AGG_TPT + k * _CH, _CH)])
    pltpu.sync_copy(zbuf.at[pl.ds(0, 120)],
                    agg_sh.at[pl.ds(s * _AGG_TPT + 512, 120)])

    def _stage(m_hbm):
        for t in range(5):
            ci = s + 16 * t

            @pl.when(ci < 78)
            def _():
                pltpu.sync_copy(m_hbm.at[pl.ds(ci * _CH, _CH)], rows_v.at[1])
                pltpu.sync_copy(rows_v.at[1], m_sh.at[pl.ds(ci * _CH, _CH)])

        @pl.when(s == 15)
        def _():
            pltpu.sync_copy(m_hbm.at[pl.ds(9984, 16)],
                            rows_v.at[1].at[pl.ds(0, 16)])
            pltpu.sync_copy(rows_v.at[1].at[pl.ds(0, 16)],
                            m_sh.at[pl.ds(9984, 16)])

    @pl.when(c == 0)
    def _():
        _stage(ml_hbm)

    @pl.when(c == 1)
    def _():
        _stage(mr_hbm)

    plsc.subcore_barrier()

    def _gwait(b):
        pltpu.make_async_copy(m_sh.at[idx_s.at[0]],
                              rows_v.at[b], gsems[b]).wait()

    def _swait(b):
        pltpu.make_async_copy(rows_v.at[b],
                              agg_sh.at[idx_d.at[0]], ssems[b]).wait()

    for half in range(2):
        pltpu.sync_copy(src_hbm.at[pl.ds(s * 160 + half * 80, 80)], idx_s)
        pltpu.sync_copy(dst_hbm.at[pl.ds(s * 160 + half * 80, 80)], idx_d)
        for b in range(2):
            pltpu.async_copy(m_sh.at[idx_s.at[b]], rows_v.at[b], gsems[b])

        def _edge(t, carry):
            g0 = t * 2
            for b in range(2):
                _gwait(b)
                pltpu.async_copy(rows_v.at[b], agg_sh.at[idx_d.at[g0 + b]],
                                 ssems[b], add=True)
            for b in range(2):
                @pl.when(g0 + b + 2 < 80)
                def _(b=b):
                    _swait(b)
                    pltpu.async_copy(m_sh.at[idx_s.at[g0 + b + 2]],
                                     rows_v.at[b], gsems[b])
            return carry

        lax.fori_loop(0, 40, _edge, 0)
        for b in range(2):
            _swait(b)

    plsc.subcore_barrier()

    def _wb(out_hbm):
        for k in range(4):
            buf = rows_v.at[k]
            pltpu.sync_copy(agg_sh.at[pl.ds(s * _AGG_TPT + k * _CH, _CH)],
                            buf)
            pltpu.sync_copy(buf, out_hbm.at[pl.ds(s * _AGG_TPT + k * _CH,
                                                  _CH)])
        pltpu.sync_copy(agg_sh.at[pl.ds(s * _AGG_TPT + 512, 120)],
                        rows_v.at[0].at[pl.ds(0, 120)])
        pltpu.sync_copy(rows_v.at[0].at[pl.ds(0, 120)],
                        out_hbm.at[pl.ds(s * _AGG_TPT + 512, 120)])

    @pl.when(c == 0)
    def _():
        _wb(aggl_hbm)

    @pl.when(c == 1)
    def _():
        _wb(aggr_hbm)


def _sc_scatter(*args):
    return pl.kernel(
        _scatter_body,
        out_type=(jax.ShapeDtypeStruct((_AGG_ROWS, 64), jnp.float32),
                  jax.ShapeDtypeStruct((_AGG_ROWS, 64), jnp.float32)),
        mesh=_sc_mesh(),
        scratch_types=[
            pltpu.VMEM((80, _CH), jnp.int32),
            pltpu.VMEM((80, _CH), jnp.int32),
            pltpu.VMEM((2, _CH, 64), jnp.float32),
            pltpu.VMEM_SHARED((_AGG_ROWS, 64), jnp.float32),
            pltpu.VMEM_SHARED((_AGG_ROWS, 64), jnp.float32),
            [pltpu.SemaphoreType.DMA] * 2,
            [pltpu.SemaphoreType.DMA] * 2,
        ],
        compiler_params=pltpu.CompilerParams(needs_layout_passes=False,
                                             use_tc_tiling_on_sc=False),
    )(*args)


_BR = 400


def _t0_body(hp_ref, norm_ref):
    deg = jnp.sum(hp_ref[...], axis=0)
    norm_ref[...] = lax.rsqrt(jnp.maximum(deg, 1.0))


def _t0(histp3):
    return pl.pallas_call(
        _t0_body,
        grid=(1,),
        in_specs=[pl.BlockSpec((32, _HIST_ROWS // _D, _D),
                               lambda i: (0, 0, 0))],
        out_specs=pl.BlockSpec((_HIST_ROWS // _D, _D), lambda i: (0, 0)),
        out_shape=jax.ShapeDtypeStruct((_HIST_ROWS // _D, _D), jnp.float32),
    )(histp3)


def _t1_body(h0_ref, no_ref, w_ref, ml_ref, mr_ref):
    hw = jnp.dot(h0_ref[...], w_ref[...],
                 preferred_element_type=jnp.float32, precision=_HIGH)
    m = hw * no_ref[...]
    ml_ref[...] = m[:, :64]
    mr_ref[...] = m[:, 64:]


_half_out_specs = [
    pl.BlockSpec((_BR, 64), lambda i: (i, 0)),
    pl.BlockSpec((_BR, 64), lambda i: (i, 0)),
]
_half_out_shape = [
    jax.ShapeDtypeStruct((_N, 64), jnp.float32),
    jax.ShapeDtypeStruct((_N, 64), jnp.float32),
]


def _t1(h0, normo, w):
    return pl.pallas_call(
        _t1_body,
        grid=(_N // _BR,),
        in_specs=[
            pl.BlockSpec((_BR, _D), lambda i: (i, 0)),
            pl.BlockSpec((_BR, 1), lambda i: (i, 0)),
            pl.BlockSpec((_D, _D), lambda i: (0, 0)),
        ],
        out_specs=_half_out_specs,
        out_shape=_half_out_shape,
    )(h0, normo, w)


def _t2_body(pl_ref, pr_ref, ni_ref, no_ref, b_ref, w_ref, ml_ref, mr_ref):
    agg = jnp.concatenate([pl_ref[...], pr_ref[...]], axis=-1)
    h1 = agg * ni_ref[...] + b_ref[...]
    hw = jnp.dot(h1, w_ref[...],
                 preferred_element_type=jnp.float32, precision=_HIGH)
    m = hw * no_ref[...]
    ml_ref[...] = m[:, :64]
    mr_ref[...] = m[:, 64:]


def _t2(aggl, aggr, ni, no, b2d, w):
    return pl.pallas_call(
        _t2_body,
        grid=(_N // _BR,),
        in_specs=[
            pl.BlockSpec((_BR, 64), lambda i: (i, 0)),
            pl.BlockSpec((_BR, 64), lambda i: (i, 0)),
            pl.BlockSpec((_BR, 1), lambda i: (i, 0)),
            pl.BlockSpec((_BR, 1), lambda i: (i, 0)),
            pl.BlockSpec((1, _D), lambda i: (0, 0)),
            pl.BlockSpec((_D, _D), lambda i: (0, 0)),
        ],
        out_specs=_half_out_specs,
        out_shape=_half_out_shape,
    )(aggl, aggr, ni, no, b2d, w)


def _ln(x, g, b):
    mu = jnp.mean(x, axis=-1, keepdims=True)
    var = jnp.mean((x - mu) ** 2, axis=-1, keepdims=True)
    return (x - mu) * lax.rsqrt(var + 1e-5) * g + b


def _t3_body(pl_ref, pr_ref, ni_ref, b_ref, g1_ref, b1_ref, w2_ref, b2_ref,
             g2_ref, b2b_ref, w3_ref, b3_ref, out_ref, acc):
    i = pl.program_id(0)
    agg = jnp.concatenate([pl_ref[...], pr_ref[...]], axis=-1)
    h2 = agg * ni_ref[...] + b_ref[...]
    bm = jnp.max(h2, axis=0, keepdims=True)

    @pl.when(i == 0)
    def _():
        acc[...] = jnp.full((8, _D), -jnp.inf, jnp.float32)

    acc[...] = jnp.maximum(acc[...], jnp.broadcast_to(bm, (8, _D)))

    @pl.when(i == pl.num_programs(0) - 1)
    def _():
        x = _ln(acc[0:1, :], g1_ref[...], b1_ref[...])
        y = jnp.dot(x, w2_ref[...],
                    preferred_element_type=jnp.float32, precision=_HIGH)
        y = jnp.maximum(y + b2_ref[...], 0.0)
        y = _ln(y, g2_ref[...], b2b_ref[...])
        out_ref[...] = (jnp.sum(y * w3_ref[...], axis=-1, keepdims=True)
                        + b3_ref[...])


def _t3(aggl, aggr, ni, b2d, g1, b1, w2, b2, g2, b2b, w3t, b3p):
    def full(shape):
        return pl.BlockSpec(shape, lambda i: tuple(0 for _ in shape))

    return pl.pallas_call(
        _t3_body,
        grid=(_N // _BR,),
        in_specs=[
            pl.BlockSpec((_BR, 64), lambda i: (i, 0)),
            pl.BlockSpec((_BR, 64), lambda i: (i, 0)),
            pl.BlockSpec((_BR, 1), lambda i: (i, 0)),
            full((1, _D)), full((1, _D)), full((1, _D)),
            full((_D, _H)), full((1, _H)), full((1, _H)), full((1, _H)),
            full((1, _H)), full((1, _D)),
        ],
        out_specs=pl.BlockSpec((1, _D), lambda i: (0, 0)),
        out_shape=jax.ShapeDtypeStruct((1, _D), jnp.float32),
        scratch_shapes=[pltpu.VMEM((8, _D), jnp.float32)],
        compiler_params=pltpu.CompilerParams(
            dimension_semantics=("arbitrary",)),
    )(aggl, aggr, ni, b2d, g1, b1, w2, b2, g2, b2b, w3t, b3p)


def kernel(node_ids, edge_index, emb, W, b, ln1_g, ln1_b, W2, b2,
           ln2_g, ln2_b, W3, b3):
    node_ids = node_ids.astype(jnp.int32)
    src = edge_index[0].astype(jnp.int32)
    dst = edge_index[1].astype(jnp.int32)

    ids2d = jnp.concatenate(
        [node_ids, jnp.zeros((_NPAD - _N,), jnp.int32)]).reshape(-1, _CH)
    src2d = jnp.concatenate(
        [src, jnp.zeros((_EPAD - _E,), jnp.int32)]).reshape(-1, _CH)
    dst2d = jnp.concatenate(
        [dst, jnp.full((_EPAD - _E,), _N, jnp.int32)]).reshape(-1, _CH)
    didx = jnp.concatenate(
        [src, dst + _N,
         jnp.full((_DEGPAD - 2 * _E,), 2 * _N, jnp.int32)])

    h0p, histp = _embed_deg(ids2d, didx, emb)
    norm = _t0(histp.reshape(32, _HIST_ROWS // _D, _D))
    nf = norm.reshape(_HIST_ROWS)
    normo = nf[:_N].reshape(_N, 1)
    normi = nf[_N:2 * _N].reshape(_N, 1)

    m1l, m1r = _t1(h0p, normo, W)
    p1l, p1r = _sc_scatter(m1l, m1r, src2d, dst2d)
    b2d = b.reshape(1, _D)
    m2l, m2r = _t2(p1l, p1r, normi, normo, b2d, W)
    p2l, p2r = _sc_scatter(m2l, m2r, src2d, dst2d)
    outp = _t3(p2l, p2r, normi, b2d,
               ln1_g.reshape(1, _D), ln1_b.reshape(1, _D),
               W2, b2.reshape(1, _H),
               ln2_g.reshape(1, _H), ln2_b.reshape(1, _H),
               W3.reshape(1, _H),
               jnp.broadcast_to(b3.reshape(1, 1), (1, _D)))
    return outp[0:1, 0:1]


# speedup vs baseline: 1.5869x; 1.0428x over previous
import functools

import jax
import jax.numpy as jnp
from jax import lax
from jax.experimental import pallas as pl
from jax.experimental.pallas import tpu as pltpu
from jax.experimental.pallas import tpu_sc as plsc

_N = 10000
_E = 320000
_D = 128
_H = 64

_CH = 128
_NPAD = 10240
_AGG_ROWS = 10112
_HIST_ROWS = 20096
_EPAD = 327680
_DEGPAD = 655360

_AGG_TPT = _AGG_ROWS // 16
_HIST_TPT = _HIST_ROWS // 16

_HIGH = jax.lax.Precision.HIGHEST


def _embed_deg_body(ids_hbm, didx_hbm, emb_hbm,
                    h0_hbm, histp_hbm,
                    idx_v, rows_v, didx_v, hist1d, gsem):
    c = lax.axis_index("c")
    s = lax.axis_index("s")
    wid = s * 2 + c

    pltpu.sync_copy(didx_hbm.at[pl.ds((c * 16 + s) * 20480, 20480)], didx_v)

    zero16 = jnp.zeros((16,), jnp.float32)

    def _z(r, carry):
        hist1d[pl.ds(r * 16, 16)] = zero16
        return carry

    lax.fori_loop(0, _HIST_ROWS // 16, _z, 0)

    for t in range(3):
        g = wid + 32 * t

        @pl.when(g < _NPAD // _CH)
        def _():
            pltpu.sync_copy(ids_hbm.at[g], idx_v)
            pltpu.async_copy(emb_hbm.at[idx_v], rows_v, gsem).wait()
            pltpu.sync_copy(rows_v, h0_hbm.at[pl.ds(g * _CH, _CH)])

    ones16 = jnp.ones((16,), jnp.float32)

    def _deg(i, carry):
        idxv = didx_v[pl.ds(i * 16, 16)]
        plsc.addupdate_scatter(hist1d, [idxv], ones16)
        return carry

    lax.fori_loop(0, 20480 // 16, _deg, 0)
    pltpu.sync_copy(hist1d, histp_hbm.at[wid])


@functools.lru_cache(maxsize=None)
def _sc_mesh():
    return plsc.VectorSubcoreMesh(core_axis_name="c", subcore_axis_name="s")


def _embed_deg(*args):
    return pl.kernel(
        _embed_deg_body,
        out_type=(jax.ShapeDtypeStruct((_NPAD, _D), jnp.float32),
                  jax.ShapeDtypeStruct((32, _HIST_ROWS), jnp.float32)),
        mesh=_sc_mesh(),
        scratch_types=[
            pltpu.VMEM((_CH,), jnp.int32),
            pltpu.VMEM((_CH, _D), jnp.float32),
            pltpu.VMEM((20480,), jnp.int32),
            pltpu.VMEM((_HIST_ROWS,), jnp.float32),
            pltpu.SemaphoreType.DMA,
        ],
        compiler_params=pltpu.CompilerParams(needs_layout_passes=False),
    )(*args)


def _scatter_body(ml_hbm, mr_hbm, src_hbm, dst_hbm,
                  aggl_hbm, aggr_hbm,
                  idx_s, idx_d, rows_v, m_sh, agg_sh, gsems, ssems):
    c = lax.axis_index("c")
    s = lax.axis_index("s")

    zero16 = jnp.zeros((16,), jnp.float32)
    zbuf = rows_v.at[0]

    def _z(r, carry):
        for k in range(4):
            zbuf[r, pl.ds(k * 16, 16)] = zero16
        return carry

    lax.fori_loop(0, _CH, _z, 0)
    for k in range(4):
        pltpu.sync_copy(zbuf,
                        agg_sh.at[pl.ds(s * _AGG_TPT + k * _CH, _CH)])
    pltpu.sync_copy(zbuf.at[pl.ds(0, 120)],
                    agg_sh.at[pl.ds(s * _AGG_TPT + 512, 120)])

    def _stage(m_hbm):
        for t in range(5):
            ci = s + 16 * t

            @pl.when(ci < 78)
            def _():
                pltpu.sync_copy(m_hbm.at[pl.ds(ci * _CH, _CH)], rows_v.at[1])
                pltpu.sync_copy(rows_v.at[1], m_sh.at[pl.ds(ci * _CH, _CH)])

        @pl.when(s == 15)
        def _():
            pltpu.sync_copy(m_hbm.at[pl.ds(9984, 16)],
                            rows_v.at[1].at[pl.ds(0, 16)])
            pltpu.sync_copy(rows_v.at[1].at[pl.ds(0, 16)],
                            m_sh.at[pl.ds(9984, 16)])

    @pl.when(c == 0)
    def _():
        _stage(ml_hbm)

    @pl.when(c == 1)
    def _():
        _stage(mr_hbm)

    plsc.subcore_barrier()

    def _gwait(b):
        pltpu.make_async_copy(m_sh.at[idx_s.at[0]],
                              rows_v.at[b], gsems[b]).wait()

    def _swait(b):
        pltpu.make_async_copy(rows_v.at[b],
                              agg_sh.at[idx_d.at[0]], ssems[b]).wait()

    for half in range(2):
        pltpu.sync_copy(src_hbm.at[pl.ds(s * 160 + half * 80, 80)], idx_s)
        pltpu.sync_copy(dst_hbm.at[pl.ds(s * 160 + half * 80, 80)], idx_d)
        for b in range(2):
            pltpu.async_copy(m_sh.at[idx_s.at[b]], rows_v.at[b], gsems[b])

        def _edge(t, carry):
            g0 = t * 2
            for b in range(2):
                _gwait(b)
                pltpu.async_copy(rows_v.at[b], agg_sh.at[idx_d.at[g0 + b]],
                                 ssems[b], add=True)
            for b in range(2):
                @pl.when(g0 + b + 2 < 80)
                def _(b=b):
                    _swait(b)
                    pltpu.async_copy(m_sh.at[idx_s.at[g0 + b + 2]],
                                     rows_v.at[b], gsems[b])
            return carry

        lax.fori_loop(0, 40, _edge, 0)
        for b in range(2):
            _swait(b)

    plsc.subcore_barrier()

    def _wb(out_hbm):
        for k in range(4):
            buf = rows_v.at[k]
            pltpu.sync_copy(agg_sh.at[pl.ds(s * _AGG_TPT + k * _CH, _CH)],
                            buf)
            pltpu.sync_copy(buf, out_hbm.at[pl.ds(s * _AGG_TPT + k * _CH,
                                                  _CH)])
        pltpu.sync_copy(agg_sh.at[pl.ds(s * _AGG_TPT + 512, 120)],
                        rows_v.at[0].at[pl.ds(0, 120)])
        pltpu.sync_copy(rows_v.at[0].at[pl.ds(0, 120)],
                        out_hbm.at[pl.ds(s * _AGG_TPT + 512, 120)])

    @pl.when(c == 0)
    def _():
        _wb(aggl_hbm)

    @pl.when(c == 1)
    def _():
        _wb(aggr_hbm)


def _sc_scatter(*args):
    return pl.kernel(
        _scatter_body,
        out_type=(jax.ShapeDtypeStruct((_AGG_ROWS, 64), jnp.float32),
                  jax.ShapeDtypeStruct((_AGG_ROWS, 64), jnp.float32)),
        mesh=_sc_mesh(),
        scratch_types=[
            pltpu.VMEM((80, _CH), jnp.int32),
            pltpu.VMEM((80, _CH), jnp.int32),
            pltpu.VMEM((2, _CH, 64), jnp.float32),
            pltpu.VMEM_SHARED((_AGG_ROWS, 64), jnp.float32),
            pltpu.VMEM_SHARED((_AGG_ROWS, 64), jnp.float32),
            [pltpu.SemaphoreType.DMA] * 2,
            [pltpu.SemaphoreType.DMA] * 2,
        ],
        compiler_params=pltpu.CompilerParams(needs_layout_passes=False,
                                             use_tc_tiling_on_sc=False),
    )(*args)


_BR = 1000


def _t0_body(hp_ref, norm_ref):
    deg = jnp.sum(hp_ref[...], axis=0)
    norm_ref[...] = lax.rsqrt(jnp.maximum(deg, 1.0))


def _t0(histp3):
    return pl.pallas_call(
        _t0_body,
        grid=(1,),
        in_specs=[pl.BlockSpec((32, _HIST_ROWS // _D, _D),
                               lambda i: (0, 0, 0))],
        out_specs=pl.BlockSpec((_HIST_ROWS // _D, _D), lambda i: (0, 0)),
        out_shape=jax.ShapeDtypeStruct((_HIST_ROWS // _D, _D), jnp.float32),
    )(histp3)


def _t1_body(h0_ref, no_ref, w_ref, ml_ref, mr_ref):
    hw = jnp.dot(h0_ref[...], w_ref[...],
                 preferred_element_type=jnp.float32, precision=_HIGH)
    m = hw * no_ref[...]
    ml_ref[...] = m[:, :64]
    mr_ref[...] = m[:, 64:]


_half_out_specs = [
    pl.BlockSpec((_BR, 64), lambda i: (i, 0)),
    pl.BlockSpec((_BR, 64), lambda i: (i, 0)),
]
_half_out_shape = [
    jax.ShapeDtypeStruct((_N, 64), jnp.float32),
    jax.ShapeDtypeStruct((_N, 64), jnp.float32),
]


def _t1(h0, normo, w):
    return pl.pallas_call(
        _t1_body,
        grid=(_N // _BR,),
        in_specs=[
            pl.BlockSpec((_BR, _D), lambda i: (i, 0)),
            pl.BlockSpec((_BR, 1), lambda i: (i, 0)),
            pl.BlockSpec((_D, _D), lambda i: (0, 0)),
        ],
        out_specs=_half_out_specs,
        out_shape=_half_out_shape,
    )(h0, normo, w)


def _t2_body(pl_ref, pr_ref, ni_ref, no_ref, b_ref, w_ref, ml_ref, mr_ref):
    agg = jnp.concatenate([pl_ref[...], pr_ref[...]], axis=-1)
    h1 = agg * ni_ref[...] + b_ref[...]
    hw = jnp.dot(h1, w_ref[...],
                 preferred_element_type=jnp.float32, precision=_HIGH)
    m = hw * no_ref[...]
    ml_ref[...] = m[:, :64]
    mr_ref[...] = m[:, 64:]


def _t2(aggl, aggr, ni, no, b2d, w):
    return pl.pallas_call(
        _t2_body,
        grid=(_N // _BR,),
        in_specs=[
            pl.BlockSpec((_BR, 64), lambda i: (i, 0)),
            pl.BlockSpec((_BR, 64), lambda i: (i, 0)),
            pl.BlockSpec((_BR, 1), lambda i: (i, 0)),
            pl.BlockSpec((_BR, 1), lambda i: (i, 0)),
            pl.BlockSpec((1, _D), lambda i: (0, 0)),
            pl.BlockSpec((_D, _D), lambda i: (0, 0)),
        ],
        out_specs=_half_out_specs,
        out_shape=_half_out_shape,
    )(aggl, aggr, ni, no, b2d, w)


def _ln(x, g, b):
    mu = jnp.mean(x, axis=-1, keepdims=True)
    var = jnp.mean((x - mu) ** 2, axis=-1, keepdims=True)
    return (x - mu) * lax.rsqrt(var + 1e-5) * g + b


def _t3_body(pl_ref, pr_ref, ni_ref, b_ref, g1_ref, b1_ref, w2_ref, b2_ref,
             g2_ref, b2b_ref, w3_ref, b3_ref, out_ref, acc):
    i = pl.program_id(0)
    agg = jnp.concatenate([pl_ref[...], pr_ref[...]], axis=-1)
    h2 = agg * ni_ref[...] + b_ref[...]
    bm = jnp.max(h2, axis=0, keepdims=True)

    @pl.when(i == 0)
    def _():
        acc[...] = jnp.full((8, _D), -jnp.inf, jnp.float32)

    acc[...] = jnp.maximum(acc[...], jnp.broadcast_to(bm, (8, _D)))

    @pl.when(i == pl.num_programs(0) - 1)
    def _():
        x = _ln(acc[0:1, :], g1_ref[...], b1_ref[...])
        y = jnp.dot(x, w2_ref[...],
                    preferred_element_type=jnp.float32, precision=_HIGH)
        y = jnp.maximum(y + b2_ref[...], 0.0)
        y = _ln(y, g2_ref[...], b2b_ref[...])
        out_ref[...] = (jnp.sum(y * w3_ref[...], axis=-1, keepdims=True)
                        + b3_ref[...])


def _t3(aggl, aggr, ni, b2d, g1, b1, w2, b2, g2, b2b, w3t, b3p):
    def full(shape):
        return pl.BlockSpec(shape, lambda i: tuple(0 for _ in shape))

    return pl.pallas_call(
        _t3_body,
        grid=(_N // _BR,),
        in_specs=[
            pl.BlockSpec((_BR, 64), lambda i: (i, 0)),
            pl.BlockSpec((_BR, 64), lambda i: (i, 0)),
            pl.BlockSpec((_BR, 1), lambda i: (i, 0)),
            full((1, _D)), full((1, _D)), full((1, _D)),
            full((_D, _H)), full((1, _H)), full((1, _H)), full((1, _H)),
            full((1, _H)), full((1, _D)),
        ],
        out_specs=pl.BlockSpec((1, _D), lambda i: (0, 0)),
        out_shape=jax.ShapeDtypeStruct((1, _D), jnp.float32),
        scratch_shapes=[pltpu.VMEM((8, _D), jnp.float32)],
        compiler_params=pltpu.CompilerParams(
            dimension_semantics=("arbitrary",)),
    )(aggl, aggr, ni, b2d, g1, b1, w2, b2, g2, b2b, w3t, b3p)


def kernel(node_ids, edge_index, emb, W, b, ln1_g, ln1_b, W2, b2,
           ln2_g, ln2_b, W3, b3):
    node_ids = node_ids.astype(jnp.int32)
    src = edge_index[0].astype(jnp.int32)
    dst = edge_index[1].astype(jnp.int32)

    ids2d = jnp.concatenate(
        [node_ids, jnp.zeros((_NPAD - _N,), jnp.int32)]).reshape(-1, _CH)
    src2d = jnp.concatenate(
        [src, jnp.zeros((_EPAD - _E,), jnp.int32)]).reshape(-1, _CH)
    dst2d = jnp.concatenate(
        [dst, jnp.full((_EPAD - _E,), _N, jnp.int32)]).reshape(-1, _CH)
    didx = jnp.concatenate(
        [src, dst + _N,
         jnp.full((_DEGPAD - 2 * _E,), 2 * _N, jnp.int32)])

    h0p, histp = _embed_deg(ids2d, didx, emb)
    norm = _t0(histp.reshape(32, _HIST_ROWS // _D, _D))
    nf = norm.reshape(_HIST_ROWS)
    normo = nf[:_N].reshape(_N, 1)
    normi = nf[_N:2 * _N].reshape(_N, 1)

    m1l, m1r = _t1(h0p, normo, W)
    p1l, p1r = _sc_scatter(m1l, m1r, src2d, dst2d)
    b2d = b.reshape(1, _D)
    m2l, m2r = _t2(p1l, p1r, normi, normo, b2d, W)
    p2l, p2r = _sc_scatter(m2l, m2r, src2d, dst2d)
    outp = _t3(p2l, p2r, normi, b2d,
               ln1_g.reshape(1, _D), ln1_b.reshape(1, _D),
               W2, b2.reshape(1, _H),
               ln2_g.reshape(1, _H), ln2_b.reshape(1, _H),
               W3.reshape(1, _H),
               jnp.broadcast_to(b3.reshape(1, 1), (1, _D)))
    return outp[0:1, 0:1]
